# baseline (speedup 1.0000x reference)
import jax
import jax.numpy as jnp
from jax.experimental import pallas as pl

H = 32
W = 32
D = 128
BS = 16


BPB = 8


def _body(row_ref, col_ref, out_ref):
    col32 = col_ref[...]
    row32 = row_ref[...]
    colB = jnp.broadcast_to(col32[None, :, :], (H, W, D))
    rowB = jnp.broadcast_to(row32[:, None, :], (H, W, D))
    img = jnp.concatenate([colB, rowB], axis=-1)
    out_ref[...] = jnp.broadcast_to(img[None], (BPB, H, W, 2 * D))


@jax.jit
def _pos_embed(row_embed, col_embed):
    out = pl.pallas_call(
        _body,
        grid=(BS // BPB,),
        in_specs=[
            pl.BlockSpec((H, D), lambda b: (0, 0)),
            pl.BlockSpec((H, D), lambda b: (0, 0)),
        ],
        out_specs=pl.BlockSpec((BPB, H, W, 2 * D), lambda b: (b, 0, 0, 0)),
        out_shape=jax.ShapeDtypeStruct((BS, H, W, 2 * D), jnp.float32),
    )(row_embed, col_embed)
    return jnp.transpose(out, (0, 3, 1, 2))


def kernel(mask, row_embed, col_embed):
    del mask
    return _pos_embed(row_embed, col_embed)


# speedup vs baseline: 1.0429x; 1.0429x over previous
import jax
import jax.numpy as jnp
from jax.experimental import pallas as pl

H = 32
W = 32
D = 128
BS = 16


BPB = 2


def _body(row_ref, col_ref, out_ref):
    col32 = col_ref[...]
    row32 = row_ref[...]
    colB = jnp.broadcast_to(col32[None, :, :], (H, W, D))
    rowB = jnp.broadcast_to(row32[:, None, :], (H, W, D))
    img = jnp.concatenate([colB, rowB], axis=-1)
    out_ref[...] = jnp.broadcast_to(img[None], (BPB, H, W, 2 * D))


@jax.jit
def _pos_embed(row_embed, col_embed):
    out = pl.pallas_call(
        _body,
        grid=(BS // BPB,),
        in_specs=[
            pl.BlockSpec((H, D), lambda b: (0, 0)),
            pl.BlockSpec((H, D), lambda b: (0, 0)),
        ],
        out_specs=pl.BlockSpec((BPB, H, W, 2 * D), lambda b: (b, 0, 0, 0)),
        out_shape=jax.ShapeDtypeStruct((BS, H, W, 2 * D), jnp.float32),
    )(row_embed, col_embed)
    return jnp.transpose(out, (0, 3, 1, 2))


def kernel(mask, row_embed, col_embed):
    del mask
    return _pos_embed(row_embed, col_embed)
